# Initial kernel scaffold; baseline (speedup 1.0000x reference)
import jax
import jax.numpy as jnp
from jax.experimental import pallas as pl


def kernel(similarity_matrix, W, att_src, att_dst, bias):
    raise NotImplementedError("write your pallas kernel here")


# speedup vs baseline: 1669.2659x; 1669.2659x over previous
import jax
import jax.numpy as jnp
from jax.experimental import pallas as pl

_N = 400


def _gat_dense_kernel(x_ref, w_ref, asrc_ref, adst_ref, bias_ref, out_ref):
    n = x_ref.shape[0]
    h = jnp.dot(x_ref[:], w_ref[:], preferred_element_type=jnp.float32)

    a_src = jax.lax.dot_general(
        asrc_ref[:], h, (((1,), (1,)), ((), ())),
        preferred_element_type=jnp.float32)
    a_dst = jax.lax.dot_general(
        h, adst_ref[:], (((1,), (1,)), ((), ())),
        preferred_element_type=jnp.float32)

    e = a_dst + a_src
    e = jnp.where(e >= 0.0, e, 0.2 * e)

    row = jax.lax.broadcasted_iota(jnp.int32, (n, n), 0)
    col = jax.lax.broadcasted_iota(jnp.int32, (n, n), 1)
    diag = row == col

    e = jnp.where(diag, -jnp.inf, e)
    m = jnp.max(e, axis=1, keepdims=True)
    p = jnp.exp(e - m)
    denom = jnp.sum(p, axis=1, keepdims=True)
    alpha = p / (denom + 1e-16)

    out = jnp.dot(alpha, h, preferred_element_type=jnp.float32) + bias_ref[:]
    out = (out + out.T) * 0.5
    out = jax.nn.sigmoid(out)
    out_ref[:] = jnp.where(diag, 1.0, out)


def kernel(similarity_matrix, W, att_src, att_dst, bias):
    asrc = att_src.reshape(1, _N)
    adst = att_dst.reshape(1, _N)
    b = bias.reshape(1, _N)
    return pl.pallas_call(
        _gat_dense_kernel,
        out_shape=jax.ShapeDtypeStruct((_N, _N), jnp.float32),
    )(similarity_matrix, W, asrc, adst, b)
